# baseline (speedup 1.0000x reference)
import functools

import jax
import jax.numpy as jnp
from jax import lax
from jax.experimental import pallas as pl
from jax.experimental.pallas import tpu as pltpu
from jax.experimental.pallas import tpu_sc as plsc

_TOP_K = 2
_TM = 256
_BT_ROUTER = 512
_BT_ADD = 512
_GCHUNK = 32
_NUM_WORKERS = 32
_NUM_SPLITS = 2


def _lane_cumsum(v):
    bt = v.shape[1]
    k = 1
    while k < bt:
        shifted = jnp.concatenate(
            [jnp.zeros(v.shape[:1] + (k,), v.dtype), v[:, :bt - k]], axis=1)
        v = v + shifted
        k *= 2
    return v


def _router_body(x_ref, wr_ref, e0_ref, e1_ref, g0_ref, g1_ref,
                 r0_ref, r1_ref, cnt_ref, carry_ref):
    i = pl.program_id(0)

    @pl.when(i == 0)
    def _():
        carry_ref[...] = jnp.zeros_like(carry_ref)

    x = x_ref[...]
    wr = wr_ref[...]
    logits_t = lax.dot_general(
        wr, x, (((0,), (1,)), ((), ())),
        precision=lax.Precision.DEFAULT,
        preferred_element_type=jnp.float32)
    ne, bt = logits_t.shape
    iota_e = lax.broadcasted_iota(jnp.int32, (ne, bt), 0)
    m1 = jnp.max(logits_t, axis=0, keepdims=True)
    i1 = jnp.min(jnp.where(logits_t == m1, iota_e, ne), axis=0,
                 keepdims=True)
    oh0 = iota_e == i1
    masked = jnp.where(oh0, -jnp.inf, logits_t)
    m2 = jnp.max(masked, axis=0, keepdims=True)
    i2 = jnp.min(jnp.where(masked == m2, iota_e, ne), axis=0,
                 keepdims=True)
    oh1 = iota_e == i2
    a = jnp.exp(m2 - m1)
    denom = 1.0 + a

    ohc = (jnp.logical_or(oh0, oh1)).astype(jnp.int32)
    inc = _lane_cumsum(ohc) + carry_ref[:, 0:1]
    r0_ref[...] = (jnp.sum(jnp.where(oh0, inc, 0), axis=0,
                           keepdims=True) - 1)[None]
    r1_ref[...] = (jnp.sum(jnp.where(oh1, inc, 0), axis=0,
                           keepdims=True) - 1)[None]
    tail = lax.slice(inc, (0, bt - 1), (ne, bt))
    carry_ref[:, 0:1] = tail
    cnt_ref[...] = tail[None]

    e0_ref[...] = i1[None]
    e1_ref[...] = i2[None]
    g0_ref[...] = (1.0 / denom)[None]
    g1_ref[...] = (a / denom)[None]


def _run_router(x, w_router):
    t, h = x.shape
    e = w_router.shape[1]
    bt = _BT_ROUTER
    nb = t // bt
    row_spec = pl.BlockSpec((1, 1, bt), lambda i: (i, 0, 0))
    return pl.pallas_call(
        _router_body,
        grid=(nb,),
        in_specs=[
            pl.BlockSpec((bt, h), lambda i: (i, 0)),
            pl.BlockSpec((h, e), lambda i: (0, 0)),
        ],
        out_specs=[row_spec] * 6 + [pl.BlockSpec((1, e, 1), lambda i: (i, 0, 0))],
        out_shape=[
            jax.ShapeDtypeStruct((nb, 1, bt), jnp.int32),
            jax.ShapeDtypeStruct((nb, 1, bt), jnp.int32),
            jax.ShapeDtypeStruct((nb, 1, bt), jnp.float32),
            jax.ShapeDtypeStruct((nb, 1, bt), jnp.float32),
            jax.ShapeDtypeStruct((nb, 1, bt), jnp.int32),
            jax.ShapeDtypeStruct((nb, 1, bt), jnp.int32),
            jax.ShapeDtypeStruct((nb, e, 1), jnp.int32),
        ],
        scratch_shapes=[pltpu.VMEM((e, 128), jnp.int32)],
        compiler_params=pltpu.CompilerParams(
            dimension_semantics=("arbitrary",)),
    )(x, w_router)


def _plan_routing(e0, e1, r0, r1, counts, num_experts, n_rows):
    t = e0.shape[0]
    p = t * _TOP_K
    padded = ((counts + _TM - 1) // _TM) * _TM
    gstart = jnp.concatenate(
        [jnp.zeros((1,), jnp.int32), jnp.cumsum(padded)[:-1].astype(jnp.int32)])
    pos0 = (gstart[e0] + r0).astype(jnp.int32)
    pos1 = (gstart[e1] + r1).astype(jnp.int32)
    pos = jnp.stack([pos0, pos1], axis=1).reshape(-1)
    row_pair = (jnp.arange(n_rows, dtype=jnp.int32) % p).at[pos].set(
        jnp.arange(p, dtype=jnp.int32))
    row_token = row_pair // _TOP_K
    n_tiles = n_rows // _TM
    tile_start = jnp.arange(n_tiles, dtype=jnp.int32) * _TM
    gend = (gstart + padded).astype(jnp.int32)
    total = gend[-1]
    tile_expert = jnp.sum(
        (tile_start[:, None] >= gend[None, :]).astype(jnp.int32), axis=1)
    tile_expert = jnp.minimum(tile_expert, num_experts - 1).astype(jnp.int32)
    tile_valid = (tile_start < total).astype(jnp.int32)
    q = jnp.concatenate([pos0, pos1])
    return row_token, tile_expert, tile_valid, q


def _sc_gather(table, idx):
    rows = idx.shape[0]
    d = table.shape[1]
    per_w = rows // _NUM_WORKERS
    n_chunks = per_w // _GCHUNK
    mesh = plsc.VectorSubcoreMesh(core_axis_name="c", subcore_axis_name="s")

    @functools.partial(
        pl.kernel,
        mesh=mesh,
        out_type=jax.ShapeDtypeStruct((rows, d), table.dtype),
        scratch_types=[
            pltpu.VMEM((per_w,), jnp.int32),
            pltpu.VMEM((2, _GCHUNK, d), table.dtype),
            pltpu.SemaphoreType.DMA,
            pltpu.SemaphoreType.DMA,
            pltpu.SemaphoreType.DMA,
        ],
    )
    def gather_kernel(tab_hbm, idx_hbm, out_hbm, idx_v, bufs, sem_g,
                      sem_o0, sem_o1):
        wid = lax.axis_index("s") * 2 + lax.axis_index("c")
        base = wid * per_w
        pltpu.sync_copy(idx_hbm.at[pl.ds(base, per_w)], idx_v)
        sem_o = (sem_o0, sem_o1)
        out_cp = [None, None]
        for c in range(n_chunks):
            b = c % 2
            if out_cp[b] is not None:
                out_cp[b].wait()
            pltpu.async_copy(
                tab_hbm.at[idx_v.at[pl.ds(c * _GCHUNK, _GCHUNK)]],
                bufs.at[b], sem_g).wait()
            out_cp[b] = pltpu.async_copy(
                bufs.at[b], out_hbm.at[pl.ds(base + c * _GCHUNK, _GCHUNK)],
                sem_o[b])
        for cp in out_cp:
            if cp is not None:
                cp.wait()

    return gather_kernel(table, idx)


def _ffn_body(te_ref, valid_ref, xs_ref, w1_ref, b1_ref, w2_ref, b2_ref,
              o_ref):
    i = pl.program_id(0)

    @pl.when(valid_ref[i] == 1)
    def _():
        x = xs_ref[...]
        h = jnp.dot(x, w1_ref[0], precision=lax.Precision.DEFAULT,
                    preferred_element_type=jnp.float32)
        h = jnp.maximum(h + b1_ref[0], 0.0)
        acc = jnp.dot(h, w2_ref[0], precision=lax.Precision.DEFAULT,
                      preferred_element_type=jnp.float32)
        o_ref[...] = acc + b2_ref[0]


def _run_ffn(xs, w1, b1, w2, b2, tile_expert, tile_valid):
    n, h = xs.shape
    f = w1.shape[2]
    n_tiles = n // _TM
    grid_spec = pltpu.PrefetchScalarGridSpec(
        num_scalar_prefetch=2,
        grid=(n_tiles,),
        in_specs=[
            pl.BlockSpec((_TM, h), lambda i, te, tv: (i, 0)),
            pl.BlockSpec((1, h, f), lambda i, te, tv: (te[i], 0, 0)),
            pl.BlockSpec((1, 1, f), lambda i, te, tv: (te[i], 0, 0)),
            pl.BlockSpec((1, f, h), lambda i, te, tv: (te[i], 0, 0)),
            pl.BlockSpec((1, 1, h), lambda i, te, tv: (te[i], 0, 0)),
        ],
        out_specs=pl.BlockSpec((_TM, h), lambda i, te, tv: (i, 0)),
    )
    return pl.pallas_call(
        _ffn_body,
        grid_spec=grid_spec,
        out_shape=jax.ShapeDtypeStruct((n, h), jnp.float32),
        compiler_params=pltpu.CompilerParams(
            dimension_semantics=("arbitrary",)),
    )(tile_expert, tile_valid, xs, w1, b1[:, None, :], w2, b2[:, None, :])


def _pair_add_body(c0_ref, c1_ref, g0_ref, g1_ref, o_ref):
    o_ref[...] = g0_ref[...] * c0_ref[...] + g1_ref[...] * c1_ref[...]


def _run_pair_add(c, g0, g1):
    t2, h = c.shape
    t = t2 // 2
    bt = _BT_ADD
    half = t // bt
    return pl.pallas_call(
        _pair_add_body,
        grid=(t // bt,),
        in_specs=[
            pl.BlockSpec((bt, h), lambda i: (i, 0)),
            pl.BlockSpec((bt, h), lambda i: (i + half, 0)),
            pl.BlockSpec((bt, 1), lambda i: (i, 0)),
            pl.BlockSpec((bt, 1), lambda i: (i, 0)),
        ],
        out_specs=pl.BlockSpec((bt, h), lambda i: (i, 0)),
        out_shape=jax.ShapeDtypeStruct((t, h), jnp.float32),
    )(c, c, g0, g1)


def kernel(hidden_states, W_router, W1, b1, W2, b2):
    b, s, h = hidden_states.shape
    num_experts = W_router.shape[1]
    t = b * s
    x = hidden_states.reshape(t, h)

    e0o, e1o, g0o, g1o, r0o, r1o, cnto = _run_router(x, W_router)
    e0, e1 = e0o.reshape(t), e1o.reshape(t)
    r0, r1 = r0o.reshape(t), r1o.reshape(t)
    g0, g1 = g0o.reshape(t, 1), g1o.reshape(t, 1)
    nb = cnto.shape[0]

    th = t // _NUM_SPLITS
    n_rows = th * _TOP_K + num_experts * _TM
    halves = []
    for k in range(_NUM_SPLITS):
        sl = slice(k * th, (k + 1) * th)
        lo = (cnto[k * (nb // _NUM_SPLITS) - 1, :, 0] if k else
              jnp.zeros((num_experts,), jnp.int32))
        hi = cnto[(k + 1) * (nb // _NUM_SPLITS) - 1, :, 0]
        row_token, tile_expert, tile_valid, q = _plan_routing(
            e0[sl], e1[sl], r0[sl] - lo[e0[sl]], r1[sl] - lo[e1[sl]],
            hi - lo, num_experts, n_rows)
        xs = _sc_gather(x, row_token + k * th)
        ys = _run_ffn(xs, W1, b1, W2, b2, tile_expert, tile_valid)
        c = _sc_gather(ys, q)
        halves.append(_run_pair_add(c, g0[sl], g1[sl]))
    y = jnp.concatenate(halves)
    return y.reshape(b, s, h)


# speedup vs baseline: 1.2746x; 1.2746x over previous
import functools

import jax
import jax.numpy as jnp
from jax import lax
from jax.experimental import pallas as pl
from jax.experimental.pallas import tpu as pltpu
from jax.experimental.pallas import tpu_sc as plsc

_TOP_K = 2
_TM = 256
_BT_ROUTER = 512
_BT_ADD = 512
_GCHUNK = 32
_NUM_WORKERS = 32


def _lane_cumsum(v):
    bt = v.shape[1]
    k = 1
    while k < bt:
        shifted = jnp.concatenate(
            [jnp.zeros(v.shape[:1] + (k,), v.dtype), v[:, :bt - k]], axis=1)
        v = v + shifted
        k *= 2
    return v


def _router_body(x_ref, wr_ref, e0_ref, e1_ref, g0_ref, g1_ref,
                 r0_ref, r1_ref, cnt_ref, carry_ref):
    i = pl.program_id(0)

    @pl.when(i == 0)
    def _():
        carry_ref[...] = jnp.zeros_like(carry_ref)

    x = x_ref[...]
    wr = wr_ref[...]
    logits_t = lax.dot_general(
        wr, x, (((0,), (1,)), ((), ())),
        precision=lax.Precision.DEFAULT,
        preferred_element_type=jnp.float32)
    ne, bt = logits_t.shape
    iota_e = lax.broadcasted_iota(jnp.int32, (ne, bt), 0)
    m1 = jnp.max(logits_t, axis=0, keepdims=True)
    i1 = jnp.min(jnp.where(logits_t == m1, iota_e, ne), axis=0,
                 keepdims=True)
    oh0 = iota_e == i1
    masked = jnp.where(oh0, -jnp.inf, logits_t)
    m2 = jnp.max(masked, axis=0, keepdims=True)
    i2 = jnp.min(jnp.where(masked == m2, iota_e, ne), axis=0,
                 keepdims=True)
    oh1 = iota_e == i2
    a = jnp.exp(m2 - m1)
    denom = 1.0 + a

    ohc = (jnp.logical_or(oh0, oh1)).astype(jnp.int32)
    inc = _lane_cumsum(ohc) + carry_ref[:, 0:1]
    r0_ref[...] = (jnp.sum(jnp.where(oh0, inc, 0), axis=0,
                           keepdims=True) - 1)[None]
    r1_ref[...] = (jnp.sum(jnp.where(oh1, inc, 0), axis=0,
                           keepdims=True) - 1)[None]
    tail = lax.slice(inc, (0, bt - 1), (ne, bt))
    carry_ref[:, 0:1] = tail
    cnt_ref[...] = tail[None]

    e0_ref[...] = i1[None]
    e1_ref[...] = i2[None]
    g0_ref[...] = (1.0 / denom)[None]
    g1_ref[...] = (a / denom)[None]


def _run_router(x, w_router):
    t, h = x.shape
    e = w_router.shape[1]
    bt = _BT_ROUTER
    nb = t // bt
    row_spec = pl.BlockSpec((1, 1, bt), lambda i: (i, 0, 0))
    return pl.pallas_call(
        _router_body,
        grid=(nb,),
        in_specs=[
            pl.BlockSpec((bt, h), lambda i: (i, 0)),
            pl.BlockSpec((h, e), lambda i: (0, 0)),
        ],
        out_specs=[row_spec] * 6 + [pl.BlockSpec((1, e, 1), lambda i: (i, 0, 0))],
        out_shape=[
            jax.ShapeDtypeStruct((nb, 1, bt), jnp.int32),
            jax.ShapeDtypeStruct((nb, 1, bt), jnp.int32),
            jax.ShapeDtypeStruct((nb, 1, bt), jnp.float32),
            jax.ShapeDtypeStruct((nb, 1, bt), jnp.float32),
            jax.ShapeDtypeStruct((nb, 1, bt), jnp.int32),
            jax.ShapeDtypeStruct((nb, 1, bt), jnp.int32),
            jax.ShapeDtypeStruct((nb, e, 1), jnp.int32),
        ],
        scratch_shapes=[pltpu.VMEM((e, 128), jnp.int32)],
        compiler_params=pltpu.CompilerParams(
            dimension_semantics=("arbitrary",)),
    )(x, w_router)


def _plan_routing(e0, e1, r0, r1, counts, num_experts, n_rows):
    t = e0.shape[0]
    p = t * _TOP_K
    padded = ((counts + _TM - 1) // _TM) * _TM
    gstart = jnp.concatenate(
        [jnp.zeros((1,), jnp.int32), jnp.cumsum(padded)[:-1].astype(jnp.int32)])
    pos0 = (gstart[e0] + r0).astype(jnp.int32)
    pos1 = (gstart[e1] + r1).astype(jnp.int32)
    pos = jnp.stack([pos0, pos1], axis=1).reshape(-1)
    row_pair = (jnp.arange(n_rows, dtype=jnp.int32) % p).at[pos].set(
        jnp.arange(p, dtype=jnp.int32))
    row_token = row_pair // _TOP_K
    n_tiles = n_rows // _TM
    tile_start = jnp.arange(n_tiles, dtype=jnp.int32) * _TM
    gend = (gstart + padded).astype(jnp.int32)
    total = gend[-1]
    tile_expert = jnp.sum(
        (tile_start[:, None] >= gend[None, :]).astype(jnp.int32), axis=1)
    tile_expert = jnp.minimum(tile_expert, num_experts - 1).astype(jnp.int32)
    tile_valid = (tile_start < total).astype(jnp.int32)
    q = jnp.concatenate([pos0, pos1])
    return row_token, tile_expert, tile_valid, q


def _sc_gather(table, idx):
    rows = idx.shape[0]
    d = table.shape[1]
    per_w = rows // _NUM_WORKERS
    chunk = 40 if per_w % 40 == 0 else _GCHUNK
    n_chunks = per_w // chunk
    mesh = plsc.VectorSubcoreMesh(core_axis_name="c", subcore_axis_name="s")

    @functools.partial(
        pl.kernel,
        mesh=mesh,
        out_type=jax.ShapeDtypeStruct((rows, d), table.dtype),
        scratch_types=[
            pltpu.VMEM((per_w,), jnp.int32),
            pltpu.VMEM((2, chunk, d), table.dtype),
            pltpu.SemaphoreType.DMA,
            pltpu.SemaphoreType.DMA,
            pltpu.SemaphoreType.DMA,
            pltpu.SemaphoreType.DMA,
        ],
    )
    def gather_kernel(tab_hbm, idx_hbm, out_hbm, idx_v, bufs, sem_g0,
                      sem_g1, sem_o0, sem_o1):
        wid = lax.axis_index("s") * 2 + lax.axis_index("c")
        base = wid * per_w
        pltpu.sync_copy(idx_hbm.at[pl.ds(base, per_w)], idx_v)
        sem_g = (sem_g0, sem_g1)
        sem_o = (sem_o0, sem_o1)
        g_cp = [None, None]
        out_cp = [None, None]

        def issue_gather(c):
            b = c % 2
            g_cp[b] = pltpu.async_copy(
                tab_hbm.at[idx_v.at[pl.ds(c * chunk, chunk)]],
                bufs.at[b], sem_g[b])

        issue_gather(0)
        for c in range(n_chunks):
            b = c % 2
            if c + 1 < n_chunks:
                bb = (c + 1) % 2
                if out_cp[bb] is not None:
                    out_cp[bb].wait()
                issue_gather(c + 1)
            g_cp[b].wait()
            out_cp[b] = pltpu.async_copy(
                bufs.at[b], out_hbm.at[pl.ds(base + c * chunk, chunk)],
                sem_o[b])
        for cp in out_cp:
            if cp is not None:
                cp.wait()

    return gather_kernel(table, idx)


def _ffn_body(te_ref, valid_ref, xs_ref, w1_ref, b1_ref, w2_ref, b2_ref,
              o_ref):
    i = pl.program_id(0)

    @pl.when(valid_ref[i] == 1)
    def _():
        x = xs_ref[...]
        h = jnp.dot(x, w1_ref[0], precision=lax.Precision.DEFAULT,
                    preferred_element_type=jnp.float32)
        h = jnp.maximum(h + b1_ref[0], 0.0)
        acc = jnp.dot(h, w2_ref[0], precision=lax.Precision.DEFAULT,
                      preferred_element_type=jnp.float32)
        o_ref[...] = acc + b2_ref[0]


def _run_ffn(xs, w1, b1, w2, b2, tile_expert, tile_valid):
    n, h = xs.shape
    f = w1.shape[2]
    n_tiles = n // _TM
    grid_spec = pltpu.PrefetchScalarGridSpec(
        num_scalar_prefetch=2,
        grid=(n_tiles,),
        in_specs=[
            pl.BlockSpec((_TM, h), lambda i, te, tv: (i, 0)),
            pl.BlockSpec((1, h, f), lambda i, te, tv: (te[i], 0, 0)),
            pl.BlockSpec((1, 1, f), lambda i, te, tv: (te[i], 0, 0)),
            pl.BlockSpec((1, f, h), lambda i, te, tv: (te[i], 0, 0)),
            pl.BlockSpec((1, 1, h), lambda i, te, tv: (te[i], 0, 0)),
        ],
        out_specs=pl.BlockSpec((_TM, h), lambda i, te, tv: (i, 0)),
    )
    return pl.pallas_call(
        _ffn_body,
        grid_spec=grid_spec,
        out_shape=jax.ShapeDtypeStruct((n, h), jnp.float32),
        compiler_params=pltpu.CompilerParams(
            dimension_semantics=("arbitrary",)),
    )(tile_expert, tile_valid, xs, w1, b1[:, None, :], w2, b2[:, None, :])


def _pair_add_body(c0_ref, c1_ref, g0_ref, g1_ref, o_ref):
    o_ref[...] = g0_ref[...] * c0_ref[...] + g1_ref[...] * c1_ref[...]


def _run_pair_add(c, g0, g1):
    t2, h = c.shape
    t = t2 // 2
    bt = _BT_ADD
    half = t // bt
    return pl.pallas_call(
        _pair_add_body,
        grid=(t // bt,),
        in_specs=[
            pl.BlockSpec((bt, h), lambda i: (i, 0)),
            pl.BlockSpec((bt, h), lambda i: (i + half, 0)),
            pl.BlockSpec((bt, 1), lambda i: (i, 0)),
            pl.BlockSpec((bt, 1), lambda i: (i, 0)),
        ],
        out_specs=pl.BlockSpec((bt, h), lambda i: (i, 0)),
        out_shape=jax.ShapeDtypeStruct((t, h), jnp.float32),
    )(c, c, g0, g1)


def kernel(hidden_states, W_router, W1, b1, W2, b2):
    b, s, h = hidden_states.shape
    num_experts = W_router.shape[1]
    t = b * s
    n_rows = t * _TOP_K + num_experts * _TM
    x = hidden_states.reshape(t, h)

    e0o, e1o, g0o, g1o, r0o, r1o, cnto = _run_router(x, W_router)
    row_token, tile_expert, tile_valid, q = _plan_routing(
        e0o.reshape(t), e1o.reshape(t), r0o.reshape(t), r1o.reshape(t),
        cnto[-1, :, 0], num_experts, n_rows)

    xs = _sc_gather(x, row_token)
    ys = _run_ffn(xs, W1, b1, W2, b2, tile_expert, tile_valid)
    c = _sc_gather(ys, q)
    y = _run_pair_add(c, g0o.reshape(t, 1), g1o.reshape(t, 1))
    return y.reshape(b, s, h)
